# baseline (speedup 1.0000x reference)
import functools

import jax
import jax.numpy as jnp
from jax import lax
from jax.experimental import pallas as pl
from jax.experimental.pallas import tpu as pltpu
from jax.experimental.pallas import tpu_sc as plsc

_R = 0.2
_NB = 10
_NSEL = 10
_V = 4096
_F = 8192
_VB = 512

_NC = 2
_NSUB = 16
_NW = _NC * _NSUB
_VPW = _V // _NW
_NG = _VPW // 16
_FPW = _F // _NW
_NGF = _FPW // 16

_SC_PARAMS = pltpu.CompilerParams(needs_layout_passes=False)


def _stagea_sc(faces_flat, verts_flat):
    mesh = plsc.VectorSubcoreMesh(core_axis_name="c", subcore_axis_name="s",
                                  num_cores=_NC, num_subcores=_NSUB)

    def body(fc_hbm, vt_hbm, raw_hbm, cent_hbm, fc_v, vt_v, raw_v, cent_v):
        wid = lax.axis_index("s") * _NC + lax.axis_index("c")
        pltpu.sync_copy(fc_hbm.at[pl.ds(wid * (_FPW * 3), _FPW * 3)], fc_v)
        pltpu.sync_copy(vt_hbm, vt_v)
        iota16 = lax.broadcasted_iota(jnp.int32, (16,), 0)

        def group(g, carry):
            i3 = iota16 * 3 + g * 48
            f0 = plsc.load_gather(fc_v, [i3])
            f1 = plsc.load_gather(fc_v, [i3 + 1])
            f2 = plsc.load_gather(fc_v, [i3 + 2])

            def vcoords(vi):
                b = vi * 3
                return (plsc.load_gather(vt_v, [b]),
                        plsc.load_gather(vt_v, [b + 1]),
                        plsc.load_gather(vt_v, [b + 2]))

            ax, ay, az = vcoords(f0)
            bx, by, bz = vcoords(f1)
            c2x, c2y, c2z = vcoords(f2)
            e1x, e1y, e1z = bx - ax, by - ay, bz - az
            e2x, e2y, e2z = c2x - ax, c2y - ay, c2z - az
            rx = e1y * e2z - e1z * e2y
            ry = e1z * e2x - e1x * e2z
            rz = e1x * e2y - e1y * e2x
            plsc.store_scatter(raw_v, [i3], rx)
            plsc.store_scatter(raw_v, [i3 + 1], ry)
            plsc.store_scatter(raw_v, [i3 + 2], rz)
            cent_v[pl.ds(g * 16, 16)] = ((ax + bx) + c2x) / 3.0
            cent_v[pl.ds(_FPW + g * 16, 16)] = ((ay + by) + c2y) / 3.0
            cent_v[pl.ds(2 * _FPW + g * 16, 16)] = ((az + bz) + c2z) / 3.0
            return carry

        lax.fori_loop(0, _NGF, group, 0)
        pltpu.sync_copy(raw_v, raw_hbm.at[pl.ds(wid * (_FPW * 3), _FPW * 3)])
        for r in range(3):
            pltpu.sync_copy(cent_v.at[pl.ds(r * _FPW, _FPW)],
                            cent_hbm.at[pl.ds(r * _F + wid * _FPW, _FPW)])

    return pl.kernel(
        body,
        out_type=(jax.ShapeDtypeStruct((3 * _F,), jnp.float32),
                  jax.ShapeDtypeStruct((3 * _F,), jnp.float32)),
        mesh=mesh,
        compiler_params=_SC_PARAMS,
        scratch_types=[
            pltpu.VMEM((_FPW * 3,), jnp.int32),
            pltpu.VMEM((_V * 3,), jnp.float32),
            pltpu.VMEM((_FPW * 3,), jnp.float32),
            pltpu.VMEM((_FPW * 3,), jnp.float32),
        ],
    )(faces_flat, verts_flat)


def _score_topk_body(pni_ref, fnT_ref, v_ref, centT_ref, idx_ref):
    pni = pni_ref[...]
    fnT = fnT_ref[...]
    v = v_ref[...]
    centT = centT_ref[...]

    inner = jnp.dot(pni, fnT, preferred_element_type=jnp.float32)
    d = jnp.dot(v, centT, preferred_element_type=jnp.float32)
    x2 = jnp.sum(v * v, axis=1, keepdims=True)
    y2 = jnp.sum(centT * centT, axis=0, keepdims=True)
    ec2 = (x2 + y2) - 2.0 * d
    ec = jnp.sqrt(jnp.maximum(ec2, 1e-12))

    m1 = jnp.sum(ec, axis=1, keepdims=True) / _F
    ec1 = ec - m1
    m2 = jnp.sum(ec1, axis=1, keepdims=True) / _F
    cen = ec1 - m2
    var = jnp.sum(cen * cen, axis=1, keepdims=True) / (_F - 1)
    std = jnp.sqrt(var)

    scores = (1.0 - inner) + (ec1 / std) * _R

    iota_f = jax.lax.broadcasted_iota(jnp.int32, (_VB, _F), 1).astype(jnp.float32)
    lane16 = jax.lax.broadcasted_iota(jnp.int32, (_VB, 16), 1)
    out = jnp.zeros((_VB, 16), jnp.int32)
    for k in range(_NB):
        rowmin = jnp.min(scores, axis=1, keepdims=True)
        cand = jnp.where(scores == rowmin, iota_f, float(_F))
        bidx = jnp.min(cand, axis=1, keepdims=True)
        out = jnp.where(lane16 == k, bidx.astype(jnp.int32), out)
        scores = jnp.where(iota_f == bidx, jnp.inf, scores)
    idx_ref[...] = out


def _score_topk(pni, fnT, v, centT):
    grid = (_V // _VB,)
    return pl.pallas_call(
        _score_topk_body,
        grid=grid,
        in_specs=[
            pl.BlockSpec((_VB, 8), lambda i: (i, 0)),
            pl.BlockSpec((8, _F), lambda i: (0, 0)),
            pl.BlockSpec((_VB, 8), lambda i: (i, 0)),
            pl.BlockSpec((8, _F), lambda i: (0, 0)),
        ],
        out_specs=pl.BlockSpec((_VB, 16), lambda i: (i, 0)),
        out_shape=jax.ShapeDtypeStruct((_V, 16), jnp.int32),
    )(pni, fnT, v, centT)


def _stagec_sc(fi_flat, faces_flat, fn_flat, verts_flat, pni_flat):
    mesh = plsc.VectorSubcoreMesh(core_axis_name="c", subcore_axis_name="s",
                                  num_cores=_NC, num_subcores=_NSUB)
    fvec = jax.ShapeDtypeStruct((_V,), jnp.float32)
    ivec = jax.ShapeDtypeStruct((_V,), jnp.int32)

    def body(fi_hbm, fc_hbm, fn_hbm, vt_hbm, pni_hbm,
             dist_hbm, ssq_hbm, cidx_hbm, sign_hbm,
             fi_v, fc_v, fn_v, vt_v, pni_v,
             dist_s, ssq_s, cidx_s, sign_s):
        wid = lax.axis_index("s") * _NC + lax.axis_index("c")
        pltpu.sync_copy(fi_hbm.at[pl.ds(wid * (_VPW * 16), _VPW * 16)], fi_v)
        pltpu.sync_copy(fc_hbm, fc_v)
        pltpu.sync_copy(fn_hbm, fn_v)
        pltpu.sync_copy(vt_hbm, vt_v)
        pltpu.sync_copy(pni_hbm.at[pl.ds(wid * (_VPW * 3), _VPW * 3)], pni_v)

        iota16 = lax.broadcasted_iota(jnp.int32, (16,), 0)
        eps = 1e-12

        def _vert_coords(vi):
            b = vi * 3
            return (plsc.load_gather(vt_v, [b]),
                    plsc.load_gather(vt_v, [b + 1]),
                    plsc.load_gather(vt_v, [b + 2]))

        def _tri_sq(px, py, pz, f):
            f3 = f * 3
            va = plsc.load_gather(fc_v, [f3])
            vb = plsc.load_gather(fc_v, [f3 + 1])
            vc = plsc.load_gather(fc_v, [f3 + 2])
            ax, ay, az = _vert_coords(va)
            bx, by, bz = _vert_coords(vb)
            cx, cy, cz = _vert_coords(vc)
            abx, aby, abz = bx - ax, by - ay, bz - az
            acx, acy, acz = cx - ax, cy - ay, cz - az
            apx, apy, apz = px - ax, py - ay, pz - az
            d1 = (abx * apx + aby * apy) + abz * apz
            d2 = (acx * apx + acy * apy) + acz * apz
            bpx, bpy, bpz = px - bx, py - by, pz - bz
            d3 = (abx * bpx + aby * bpy) + abz * bpz
            d4 = (acx * bpx + acy * bpy) + acz * bpz
            cpx, cpy, cpz = px - cx, py - cy, pz - cz
            d5 = (abx * cpx + aby * cpy) + abz * cpz
            d6 = (acx * cpx + acy * cpy) + acz * cpz
            va_ = d3 * d6 - d5 * d4
            vb_ = d5 * d2 - d1 * d6
            vc_ = d1 * d4 - d3 * d2

            def _safe_div(num, den):
                den = jnp.where(jnp.abs(den) < eps, eps, den)
                return num / den

            def _clip01(x):
                return jnp.minimum(jnp.maximum(x, 0.0), 1.0)

            t_ab = _clip01(_safe_div(d1, d1 - d3))
            t_ac = _clip01(_safe_div(d2, d2 - d6))
            t_bc = _clip01(_safe_div(d4 - d3, (d4 - d3) + (d5 - d6)))
            inv = _safe_div(jnp.ones_like(va_), va_ + vb_ + vc_)
            vv = vb_ * inv
            ww = vc_ * inv
            clx = ax + abx * vv + acx * ww
            cly = ay + aby * vv + acy * ww
            clz = az + abz * vv + acz * ww
            c1 = (va_ <= 0) & ((d4 - d3) >= 0) & ((d5 - d6) >= 0)
            clx = jnp.where(c1, bx + (cx - bx) * t_bc, clx)
            cly = jnp.where(c1, by + (cy - by) * t_bc, cly)
            clz = jnp.where(c1, bz + (cz - bz) * t_bc, clz)
            c2 = (vb_ <= 0) & (d2 >= 0) & (d6 <= 0)
            clx = jnp.where(c2, ax + acx * t_ac, clx)
            cly = jnp.where(c2, ay + acy * t_ac, cly)
            clz = jnp.where(c2, az + acz * t_ac, clz)
            c3 = (vc_ <= 0) & (d1 >= 0) & (d3 <= 0)
            clx = jnp.where(c3, ax + abx * t_ab, clx)
            cly = jnp.where(c3, ay + aby * t_ab, cly)
            clz = jnp.where(c3, az + abz * t_ab, clz)
            c4 = (d6 >= 0) & (d5 <= d6)
            clx = jnp.where(c4, cx, clx)
            cly = jnp.where(c4, cy, cly)
            clz = jnp.where(c4, cz, clz)
            c5 = (d3 >= 0) & (d4 <= d3)
            clx = jnp.where(c5, bx, clx)
            cly = jnp.where(c5, by, cly)
            clz = jnp.where(c5, bz, clz)
            c6 = (d1 <= 0) & (d2 <= 0)
            clx = jnp.where(c6, ax, clx)
            cly = jnp.where(c6, ay, cly)
            clz = jnp.where(c6, az, clz)
            dx, dy, dz = px - clx, py - cly, pz - clz
            return (dx * dx + dy * dy) + dz * dz

        def group(g, carry):
            ip = iota16 * 3 + g * 48
            pn0 = plsc.load_gather(pni_v, [ip])
            pn1 = plsc.load_gather(pni_v, [ip + 1])
            pn2 = plsc.load_gather(pni_v, [ip + 2])
            gid = (wid * _VPW + g * 16) * 3
            iv3 = iota16 * 3 + gid
            px = plsc.load_gather(vt_v, [iv3])
            py = plsc.load_gather(vt_v, [iv3 + 1])
            pz = plsc.load_gather(vt_v, [iv3 + 2])

            ivs, fidx = [], []
            for j in range(_NB):
                idx = iota16 * 16 + (g * 256 + j)
                fj = plsc.load_gather(fi_v, [idx])
                fj3 = fj * 3
                n0 = plsc.load_gather(fn_v, [fj3])
                n1 = plsc.load_gather(fn_v, [fj3 + 1])
                n2 = plsc.load_gather(fn_v, [fj3 + 2])
                iv = (pn0 * n0 + pn1 * n1) + pn2 * n2
                iv = jnp.where(iv > 0.5, iv, -1.0)
                ivs.append(iv)
                fidx.append(fj)

            mindist = jnp.full((16,), jnp.inf, jnp.float32)
            fstar = jnp.zeros((16,), jnp.int32)
            neg_inf = jnp.full((16,), -jnp.inf, jnp.float32)
            for k in range(_NSEL):
                best = ivs[0]
                bidx = fidx[0]
                barg = jnp.zeros((16,), jnp.int32)
                for j in range(1, _NB):
                    cond = ivs[j] > best
                    best = jnp.where(cond, ivs[j], best)
                    bidx = jnp.where(cond, fidx[j], bidx)
                    barg = jnp.where(cond, j, barg)
                for j in range(_NB):
                    ivs[j] = jnp.where(barg == j, neg_inf, ivs[j])
                sq = _tri_sq(px, py, pz, bidx)
                cond2 = sq < mindist
                mindist = jnp.where(cond2, sq, mindist)
                fstar = jnp.where(cond2, bidx, fstar)

            f3 = fstar * 3
            va = plsc.load_gather(fc_v, [f3])
            vb = plsc.load_gather(fc_v, [f3 + 1])
            vc = plsc.load_gather(fc_v, [f3 + 2])
            ax, ay, az = _vert_coords(va)
            bx, by, bz = _vert_coords(vb)
            cx, cy, cz = _vert_coords(vc)
            n0 = plsc.load_gather(fn_v, [f3])
            n1 = plsc.load_gather(fn_v, [f3 + 1])
            n2 = plsc.load_gather(fn_v, [f3 + 2])
            cenx = ((ax + bx) + cx) / 3.0
            ceny = ((ay + by) + cy) / 3.0
            cenz = ((az + bz) + cz) / 3.0
            dvx, dvy, dvz = px - cenx, py - ceny, pz - cenz
            ssq = (dvx * dvx + dvy * dvy) + dvz * dvz
            sgn = -((dvx * n0 + dvy * n1) + dvz * n2)

            dist_s[pl.ds(g * 16, 16)] = mindist
            ssq_s[pl.ds(g * 16, 16)] = ssq
            cidx_s[pl.ds(g * 16, 16)] = fstar
            sign_s[pl.ds(g * 16, 16)] = sgn
            return carry

        lax.fori_loop(0, _NG, group, 0)
        pltpu.sync_copy(dist_s, dist_hbm.at[pl.ds(wid * _VPW, _VPW)])
        pltpu.sync_copy(ssq_s, ssq_hbm.at[pl.ds(wid * _VPW, _VPW)])
        pltpu.sync_copy(cidx_s, cidx_hbm.at[pl.ds(wid * _VPW, _VPW)])
        pltpu.sync_copy(sign_s, sign_hbm.at[pl.ds(wid * _VPW, _VPW)])

    return pl.kernel(
        body,
        out_type=(fvec, fvec, ivec, fvec),
        mesh=mesh,
        compiler_params=_SC_PARAMS,
        scratch_types=[
            pltpu.VMEM((_VPW * 16,), jnp.int32),
            pltpu.VMEM((_F * 3,), jnp.int32),
            pltpu.VMEM((_F * 3,), jnp.float32),
            pltpu.VMEM((_V * 3,), jnp.float32),
            pltpu.VMEM((_VPW * 3,), jnp.float32),
            pltpu.VMEM((_VPW,), jnp.float32),
            pltpu.VMEM((_VPW,), jnp.float32),
            pltpu.VMEM((_VPW,), jnp.int32),
            pltpu.VMEM((_VPW,), jnp.float32),
        ],
    )(fi_flat, faces_flat, fn_flat, verts_flat, pni_flat)


def kernel(verts, faces):
    faces = faces.astype(jnp.int32)
    verts_p = verts[0]
    faces_flat = faces.reshape(-1)
    verts_flat = verts_p.reshape(-1)

    raw_flat, cent3 = _stagea_sc(faces_flat, verts_flat)
    face_n_raw = raw_flat.reshape(_F, 3)
    faces_normals_packed = face_n_raw / jnp.maximum(
        jnp.linalg.norm(face_n_raw, axis=-1, keepdims=True), 1e-6)
    vn = jnp.zeros_like(verts_p)
    scat_idx = jnp.concatenate([faces[:, 0], faces[:, 1], faces[:, 2]])
    scat_upd = jnp.concatenate([face_n_raw, face_n_raw, face_n_raw])
    vn = vn.at[scat_idx].add(scat_upd)
    pt_normals = vn / jnp.maximum(jnp.linalg.norm(vn, axis=-1, keepdims=True), 1e-6)
    pni3 = -pt_normals

    zpadV = jnp.zeros((_V, 5), jnp.float32)
    zpadF = jnp.zeros((5, _F), jnp.float32)
    pni = jnp.concatenate([pni3, zpadV], axis=1)
    vpad = jnp.concatenate([verts_p, zpadV], axis=1)
    fnT = jnp.concatenate([faces_normals_packed.T, zpadF], axis=0)
    centT = jnp.concatenate([cent3.reshape(3, _F), zpadF], axis=0)
    fi16 = _score_topk(pni, fnT, vpad, centT)

    dist, ssq, closed_indx, sign = _stagec_sc(
        fi16.reshape(-1), faces_flat, faces_normals_packed.reshape(-1),
        verts_flat, pni3.reshape(-1))
    return dist, jnp.sqrt(ssq), closed_indx, sign


# speedup vs baseline: 1.1498x; 1.1498x over previous
import functools

import jax
import jax.numpy as jnp
from jax import lax
from jax.experimental import pallas as pl
from jax.experimental.pallas import tpu as pltpu
from jax.experimental.pallas import tpu_sc as plsc

_R = 0.2
_NB = 10
_NSEL = 10
_V = 4096
_F = 8192
_VB = 256

_NC = 2
_NSUB = 16
_NW = _NC * _NSUB
_VPW = _V // _NW
_NG = _VPW // 16
_FPW = _F // _NW
_NGF = _FPW // 16

_SC_PARAMS = pltpu.CompilerParams(needs_layout_passes=False)


def _stagea_sc(faces_flat, verts_flat):
    mesh = plsc.VectorSubcoreMesh(core_axis_name="c", subcore_axis_name="s",
                                  num_cores=_NC, num_subcores=_NSUB)

    def body(fc_hbm, vt_hbm, raw_hbm, cent_hbm, fc_v, vt_v, raw_v, cent_v):
        wid = lax.axis_index("s") * _NC + lax.axis_index("c")
        pltpu.sync_copy(fc_hbm.at[pl.ds(wid * (_FPW * 3), _FPW * 3)], fc_v)
        pltpu.sync_copy(vt_hbm, vt_v)
        iota16 = lax.broadcasted_iota(jnp.int32, (16,), 0)

        def group(g, carry):
            i3 = iota16 * 3 + g * 48
            f0 = plsc.load_gather(fc_v, [i3])
            f1 = plsc.load_gather(fc_v, [i3 + 1])
            f2 = plsc.load_gather(fc_v, [i3 + 2])

            def vcoords(vi):
                b = vi * 3
                return (plsc.load_gather(vt_v, [b]),
                        plsc.load_gather(vt_v, [b + 1]),
                        plsc.load_gather(vt_v, [b + 2]))

            ax, ay, az = vcoords(f0)
            bx, by, bz = vcoords(f1)
            c2x, c2y, c2z = vcoords(f2)
            e1x, e1y, e1z = bx - ax, by - ay, bz - az
            e2x, e2y, e2z = c2x - ax, c2y - ay, c2z - az
            rx = e1y * e2z - e1z * e2y
            ry = e1z * e2x - e1x * e2z
            rz = e1x * e2y - e1y * e2x
            plsc.store_scatter(raw_v, [i3], rx)
            plsc.store_scatter(raw_v, [i3 + 1], ry)
            plsc.store_scatter(raw_v, [i3 + 2], rz)
            cent_v[pl.ds(g * 16, 16)] = ((ax + bx) + c2x) / 3.0
            cent_v[pl.ds(_FPW + g * 16, 16)] = ((ay + by) + c2y) / 3.0
            cent_v[pl.ds(2 * _FPW + g * 16, 16)] = ((az + bz) + c2z) / 3.0
            return carry

        lax.fori_loop(0, _NGF, group, 0)
        pltpu.sync_copy(raw_v, raw_hbm.at[pl.ds(wid * (_FPW * 3), _FPW * 3)])
        for r in range(3):
            pltpu.sync_copy(cent_v.at[pl.ds(r * _FPW, _FPW)],
                            cent_hbm.at[pl.ds(r * _F + wid * _FPW, _FPW)])

    return pl.kernel(
        body,
        out_type=(jax.ShapeDtypeStruct((3 * _F,), jnp.float32),
                  jax.ShapeDtypeStruct((3 * _F,), jnp.float32)),
        mesh=mesh,
        compiler_params=_SC_PARAMS,
        scratch_types=[
            pltpu.VMEM((_FPW * 3,), jnp.int32),
            pltpu.VMEM((_V * 3,), jnp.float32),
            pltpu.VMEM((_FPW * 3,), jnp.float32),
            pltpu.VMEM((_FPW * 3,), jnp.float32),
        ],
    )(faces_flat, verts_flat)


def _score_topk_body(pni_ref, fnT_ref, v_ref, centT_ref, idx_ref):
    pni = pni_ref[...]
    fnT = fnT_ref[...]
    v = v_ref[...]
    centT = centT_ref[...]

    inner = jnp.dot(pni, fnT, preferred_element_type=jnp.float32)
    d = jnp.dot(v, centT, preferred_element_type=jnp.float32)
    x2 = jnp.sum(v * v, axis=1, keepdims=True)
    y2 = jnp.sum(centT * centT, axis=0, keepdims=True)
    ec2 = (x2 + y2) - 2.0 * d
    ec = jnp.sqrt(jnp.maximum(ec2, 1e-12))

    m1 = jnp.sum(ec, axis=1, keepdims=True) / _F
    ec1 = ec - m1
    m2 = jnp.sum(ec1, axis=1, keepdims=True) / _F
    cen = ec1 - m2
    var = jnp.sum(cen * cen, axis=1, keepdims=True) / (_F - 1)
    std = jnp.sqrt(var)

    scores = (1.0 - inner) + (ec1 / std) * _R

    iota_f = jax.lax.broadcasted_iota(jnp.int32, (_VB, _F), 1).astype(jnp.float32)
    lane16 = jax.lax.broadcasted_iota(jnp.int32, (_VB, 16), 1)
    out = jnp.zeros((_VB, 16), jnp.int32)
    def _rmin(x):
        p = [jnp.min(x[:, i * (_F // 4):(i + 1) * (_F // 4)], axis=1, keepdims=True)
             for i in range(4)]
        return jnp.minimum(jnp.minimum(p[0], p[1]), jnp.minimum(p[2], p[3]))

    for k in range(_NB):
        rowmin = _rmin(scores)
        cand = jnp.where(scores == rowmin, iota_f, float(_F))
        bidx = _rmin(cand)
        out = jnp.where(lane16 == k, bidx.astype(jnp.int32), out)
        scores = jnp.where(iota_f == bidx, jnp.inf, scores)
    idx_ref[...] = out


def _score_topk(pni, fnT, v, centT):
    grid = (_V // _VB,)
    return pl.pallas_call(
        _score_topk_body,
        grid=grid,
        in_specs=[
            pl.BlockSpec((_VB, 8), lambda i: (i, 0)),
            pl.BlockSpec((8, _F), lambda i: (0, 0)),
            pl.BlockSpec((_VB, 8), lambda i: (i, 0)),
            pl.BlockSpec((8, _F), lambda i: (0, 0)),
        ],
        out_specs=pl.BlockSpec((_VB, 16), lambda i: (i, 0)),
        out_shape=jax.ShapeDtypeStruct((_V, 16), jnp.int32),
    )(pni, fnT, v, centT)


def _stagec_sc(fi_flat, faces_flat, fn_flat, verts_flat, pni_flat):
    mesh = plsc.VectorSubcoreMesh(core_axis_name="c", subcore_axis_name="s",
                                  num_cores=_NC, num_subcores=_NSUB)
    fvec = jax.ShapeDtypeStruct((_V,), jnp.float32)
    ivec = jax.ShapeDtypeStruct((_V,), jnp.int32)

    def body(fi_hbm, fc_hbm, fn_hbm, vt_hbm, pni_hbm,
             dist_hbm, ssq_hbm, cidx_hbm, sign_hbm,
             fi_v, fc_v, fn_v, vt_v, pni_v,
             dist_s, ssq_s, cidx_s, sign_s):
        wid = lax.axis_index("s") * _NC + lax.axis_index("c")
        pltpu.sync_copy(fi_hbm.at[pl.ds(wid * (_VPW * 16), _VPW * 16)], fi_v)
        pltpu.sync_copy(fc_hbm, fc_v)
        pltpu.sync_copy(fn_hbm, fn_v)
        pltpu.sync_copy(vt_hbm, vt_v)
        pltpu.sync_copy(pni_hbm.at[pl.ds(wid * (_VPW * 3), _VPW * 3)], pni_v)

        iota16 = lax.broadcasted_iota(jnp.int32, (16,), 0)
        eps = 1e-12

        def _vert_coords(vi):
            b = vi * 3
            return (plsc.load_gather(vt_v, [b]),
                    plsc.load_gather(vt_v, [b + 1]),
                    plsc.load_gather(vt_v, [b + 2]))

        def _tri_sq(px, py, pz, f):
            f3 = f * 3
            va = plsc.load_gather(fc_v, [f3])
            vb = plsc.load_gather(fc_v, [f3 + 1])
            vc = plsc.load_gather(fc_v, [f3 + 2])
            ax, ay, az = _vert_coords(va)
            bx, by, bz = _vert_coords(vb)
            cx, cy, cz = _vert_coords(vc)
            abx, aby, abz = bx - ax, by - ay, bz - az
            acx, acy, acz = cx - ax, cy - ay, cz - az
            apx, apy, apz = px - ax, py - ay, pz - az
            d1 = (abx * apx + aby * apy) + abz * apz
            d2 = (acx * apx + acy * apy) + acz * apz
            bpx, bpy, bpz = px - bx, py - by, pz - bz
            d3 = (abx * bpx + aby * bpy) + abz * bpz
            d4 = (acx * bpx + acy * bpy) + acz * bpz
            cpx, cpy, cpz = px - cx, py - cy, pz - cz
            d5 = (abx * cpx + aby * cpy) + abz * cpz
            d6 = (acx * cpx + acy * cpy) + acz * cpz
            va_ = d3 * d6 - d5 * d4
            vb_ = d5 * d2 - d1 * d6
            vc_ = d1 * d4 - d3 * d2

            def _safe_div(num, den):
                den = jnp.where(jnp.abs(den) < eps, eps, den)
                return num / den

            def _clip01(x):
                return jnp.minimum(jnp.maximum(x, 0.0), 1.0)

            t_ab = _clip01(_safe_div(d1, d1 - d3))
            t_ac = _clip01(_safe_div(d2, d2 - d6))
            t_bc = _clip01(_safe_div(d4 - d3, (d4 - d3) + (d5 - d6)))
            inv = _safe_div(jnp.ones_like(va_), va_ + vb_ + vc_)
            vv = vb_ * inv
            ww = vc_ * inv
            clx = ax + abx * vv + acx * ww
            cly = ay + aby * vv + acy * ww
            clz = az + abz * vv + acz * ww
            c1 = (va_ <= 0) & ((d4 - d3) >= 0) & ((d5 - d6) >= 0)
            clx = jnp.where(c1, bx + (cx - bx) * t_bc, clx)
            cly = jnp.where(c1, by + (cy - by) * t_bc, cly)
            clz = jnp.where(c1, bz + (cz - bz) * t_bc, clz)
            c2 = (vb_ <= 0) & (d2 >= 0) & (d6 <= 0)
            clx = jnp.where(c2, ax + acx * t_ac, clx)
            cly = jnp.where(c2, ay + acy * t_ac, cly)
            clz = jnp.where(c2, az + acz * t_ac, clz)
            c3 = (vc_ <= 0) & (d1 >= 0) & (d3 <= 0)
            clx = jnp.where(c3, ax + abx * t_ab, clx)
            cly = jnp.where(c3, ay + aby * t_ab, cly)
            clz = jnp.where(c3, az + abz * t_ab, clz)
            c4 = (d6 >= 0) & (d5 <= d6)
            clx = jnp.where(c4, cx, clx)
            cly = jnp.where(c4, cy, cly)
            clz = jnp.where(c4, cz, clz)
            c5 = (d3 >= 0) & (d4 <= d3)
            clx = jnp.where(c5, bx, clx)
            cly = jnp.where(c5, by, cly)
            clz = jnp.where(c5, bz, clz)
            c6 = (d1 <= 0) & (d2 <= 0)
            clx = jnp.where(c6, ax, clx)
            cly = jnp.where(c6, ay, cly)
            clz = jnp.where(c6, az, clz)
            dx, dy, dz = px - clx, py - cly, pz - clz
            return (dx * dx + dy * dy) + dz * dz

        def group(g, carry):
            ip = iota16 * 3 + g * 48
            pn0 = plsc.load_gather(pni_v, [ip])
            pn1 = plsc.load_gather(pni_v, [ip + 1])
            pn2 = plsc.load_gather(pni_v, [ip + 2])
            gid = (wid * _VPW + g * 16) * 3
            iv3 = iota16 * 3 + gid
            px = plsc.load_gather(vt_v, [iv3])
            py = plsc.load_gather(vt_v, [iv3 + 1])
            pz = plsc.load_gather(vt_v, [iv3 + 2])

            ivs, fidx = [], []
            for j in range(_NB):
                idx = iota16 * 16 + (g * 256 + j)
                fj = plsc.load_gather(fi_v, [idx])
                fj3 = fj * 3
                n0 = plsc.load_gather(fn_v, [fj3])
                n1 = plsc.load_gather(fn_v, [fj3 + 1])
                n2 = plsc.load_gather(fn_v, [fj3 + 2])
                iv = (pn0 * n0 + pn1 * n1) + pn2 * n2
                iv = jnp.where(iv > 0.5, iv, -1.0)
                ivs.append(iv)
                fidx.append(fj)

            mindist = jnp.full((16,), jnp.inf, jnp.float32)
            fstar = jnp.zeros((16,), jnp.int32)
            neg_inf = jnp.full((16,), -jnp.inf, jnp.float32)
            for k in range(_NSEL):
                best = ivs[0]
                bidx = fidx[0]
                barg = jnp.zeros((16,), jnp.int32)
                for j in range(1, _NB):
                    cond = ivs[j] > best
                    best = jnp.where(cond, ivs[j], best)
                    bidx = jnp.where(cond, fidx[j], bidx)
                    barg = jnp.where(cond, j, barg)
                for j in range(_NB):
                    ivs[j] = jnp.where(barg == j, neg_inf, ivs[j])
                sq = _tri_sq(px, py, pz, bidx)
                cond2 = sq < mindist
                mindist = jnp.where(cond2, sq, mindist)
                fstar = jnp.where(cond2, bidx, fstar)

            f3 = fstar * 3
            va = plsc.load_gather(fc_v, [f3])
            vb = plsc.load_gather(fc_v, [f3 + 1])
            vc = plsc.load_gather(fc_v, [f3 + 2])
            ax, ay, az = _vert_coords(va)
            bx, by, bz = _vert_coords(vb)
            cx, cy, cz = _vert_coords(vc)
            n0 = plsc.load_gather(fn_v, [f3])
            n1 = plsc.load_gather(fn_v, [f3 + 1])
            n2 = plsc.load_gather(fn_v, [f3 + 2])
            cenx = ((ax + bx) + cx) / 3.0
            ceny = ((ay + by) + cy) / 3.0
            cenz = ((az + bz) + cz) / 3.0
            dvx, dvy, dvz = px - cenx, py - ceny, pz - cenz
            ssq = (dvx * dvx + dvy * dvy) + dvz * dvz
            sgn = -((dvx * n0 + dvy * n1) + dvz * n2)

            dist_s[pl.ds(g * 16, 16)] = mindist
            ssq_s[pl.ds(g * 16, 16)] = ssq
            cidx_s[pl.ds(g * 16, 16)] = fstar
            sign_s[pl.ds(g * 16, 16)] = sgn
            return carry

        lax.fori_loop(0, _NG, group, 0)
        pltpu.sync_copy(dist_s, dist_hbm.at[pl.ds(wid * _VPW, _VPW)])
        pltpu.sync_copy(ssq_s, ssq_hbm.at[pl.ds(wid * _VPW, _VPW)])
        pltpu.sync_copy(cidx_s, cidx_hbm.at[pl.ds(wid * _VPW, _VPW)])
        pltpu.sync_copy(sign_s, sign_hbm.at[pl.ds(wid * _VPW, _VPW)])

    return pl.kernel(
        body,
        out_type=(fvec, fvec, ivec, fvec),
        mesh=mesh,
        compiler_params=_SC_PARAMS,
        scratch_types=[
            pltpu.VMEM((_VPW * 16,), jnp.int32),
            pltpu.VMEM((_F * 3,), jnp.int32),
            pltpu.VMEM((_F * 3,), jnp.float32),
            pltpu.VMEM((_V * 3,), jnp.float32),
            pltpu.VMEM((_VPW * 3,), jnp.float32),
            pltpu.VMEM((_VPW,), jnp.float32),
            pltpu.VMEM((_VPW,), jnp.float32),
            pltpu.VMEM((_VPW,), jnp.int32),
            pltpu.VMEM((_VPW,), jnp.float32),
        ],
    )(fi_flat, faces_flat, fn_flat, verts_flat, pni_flat)


def kernel(verts, faces):
    faces = faces.astype(jnp.int32)
    verts_p = verts[0]
    faces_flat = faces.reshape(-1)
    verts_flat = verts_p.reshape(-1)

    raw_flat, cent3 = _stagea_sc(faces_flat, verts_flat)
    face_n_raw = raw_flat.reshape(_F, 3)
    faces_normals_packed = face_n_raw / jnp.maximum(
        jnp.linalg.norm(face_n_raw, axis=-1, keepdims=True), 1e-6)
    vn = jnp.zeros_like(verts_p)
    scat_idx = jnp.concatenate([faces[:, 0], faces[:, 1], faces[:, 2]])
    scat_upd = jnp.concatenate([face_n_raw, face_n_raw, face_n_raw])
    vn = vn.at[scat_idx].add(scat_upd)
    pt_normals = vn / jnp.maximum(jnp.linalg.norm(vn, axis=-1, keepdims=True), 1e-6)
    pni3 = -pt_normals

    zpadV = jnp.zeros((_V, 5), jnp.float32)
    zpadF = jnp.zeros((5, _F), jnp.float32)
    pni = jnp.concatenate([pni3, zpadV], axis=1)
    vpad = jnp.concatenate([verts_p, zpadV], axis=1)
    fnT = jnp.concatenate([faces_normals_packed.T, zpadF], axis=0)
    centT = jnp.concatenate([cent3.reshape(3, _F), zpadF], axis=0)
    fi16 = _score_topk(pni, fnT, vpad, centT)

    dist, ssq, closed_indx, sign = _stagec_sc(
        fi16.reshape(-1), faces_flat, faces_normals_packed.reshape(-1),
        verts_flat, pni3.reshape(-1))
    return dist, jnp.sqrt(ssq), closed_indx, sign


# speedup vs baseline: 1.1510x; 1.0011x over previous
import jax
import jax.numpy as jnp
from jax import lax
from jax.experimental import pallas as pl
from jax.experimental.pallas import tpu as pltpu
from jax.experimental.pallas import tpu_sc as plsc

_R = 0.2
_NB = 10
_NSEL = 10
_V = 4096
_F = 8192
_VB = 256

_NC = 2
_NSUB = 16
_NW = _NC * _NSUB
_VPW = _V // _NW
_NG = _VPW // 16
_FPW = _F // _NW
_NGF = _FPW // 16

_SC_PARAMS = pltpu.CompilerParams(needs_layout_passes=False)


def _stagea_sc(faces_flat, verts_flat):
    mesh = plsc.VectorSubcoreMesh(core_axis_name="c", subcore_axis_name="s",
                                  num_cores=_NC, num_subcores=_NSUB)

    def body(fc_hbm, vt_hbm, raw_hbm, cent_hbm, fc_v, vt_v, raw_v, cent_v):
        wid = lax.axis_index("s") * _NC + lax.axis_index("c")
        pltpu.sync_copy(fc_hbm.at[pl.ds(wid * (_FPW * 3), _FPW * 3)], fc_v)
        pltpu.sync_copy(vt_hbm, vt_v)
        iota16 = lax.broadcasted_iota(jnp.int32, (16,), 0)

        def group(g, carry):
            i3 = iota16 * 3 + g * 48
            f0 = plsc.load_gather(fc_v, [i3])
            f1 = plsc.load_gather(fc_v, [i3 + 1])
            f2 = plsc.load_gather(fc_v, [i3 + 2])

            def vcoords(vi):
                b = vi * 3
                return (plsc.load_gather(vt_v, [b]),
                        plsc.load_gather(vt_v, [b + 1]),
                        plsc.load_gather(vt_v, [b + 2]))

            ax, ay, az = vcoords(f0)
            bx, by, bz = vcoords(f1)
            c2x, c2y, c2z = vcoords(f2)
            e1x, e1y, e1z = bx - ax, by - ay, bz - az
            e2x, e2y, e2z = c2x - ax, c2y - ay, c2z - az
            rx = e1y * e2z - e1z * e2y
            ry = e1z * e2x - e1x * e2z
            rz = e1x * e2y - e1y * e2x
            plsc.store_scatter(raw_v, [i3], rx)
            plsc.store_scatter(raw_v, [i3 + 1], ry)
            plsc.store_scatter(raw_v, [i3 + 2], rz)
            cent_v[pl.ds(g * 16, 16)] = ((ax + bx) + c2x) / 3.0
            cent_v[pl.ds(_FPW + g * 16, 16)] = ((ay + by) + c2y) / 3.0
            cent_v[pl.ds(2 * _FPW + g * 16, 16)] = ((az + bz) + c2z) / 3.0
            return carry

        lax.fori_loop(0, _NGF, group, 0)
        pltpu.sync_copy(raw_v, raw_hbm.at[pl.ds(wid * (_FPW * 3), _FPW * 3)])
        for r in range(3):
            pltpu.sync_copy(cent_v.at[pl.ds(r * _FPW, _FPW)],
                            cent_hbm.at[pl.ds(r * _F + wid * _FPW, _FPW)])

    return pl.kernel(
        body,
        out_type=(jax.ShapeDtypeStruct((3 * _F,), jnp.float32),
                  jax.ShapeDtypeStruct((3 * _F,), jnp.float32)),
        mesh=mesh,
        compiler_params=_SC_PARAMS,
        scratch_types=[
            pltpu.VMEM((_FPW * 3,), jnp.int32),
            pltpu.VMEM((_V * 3,), jnp.float32),
            pltpu.VMEM((_FPW * 3,), jnp.float32),
            pltpu.VMEM((_FPW * 3,), jnp.float32),
        ],
    )(faces_flat, verts_flat)


def _score_topk_body(pni_ref, fnT_ref, v_ref, centT_ref, idx_ref):
    pni = pni_ref[...]
    fnT = fnT_ref[...]
    v = v_ref[...]
    centT = centT_ref[...]

    inner = jnp.dot(pni, fnT, preferred_element_type=jnp.float32)
    d = jnp.dot(v, centT, preferred_element_type=jnp.float32)
    x2 = jnp.sum(v * v, axis=1, keepdims=True)
    y2 = jnp.sum(centT * centT, axis=0, keepdims=True)
    ec2 = (x2 + y2) - 2.0 * d
    ec = jnp.sqrt(jnp.maximum(ec2, 1e-12))

    m1 = jnp.sum(ec, axis=1, keepdims=True) / _F
    ec1 = ec - m1
    m2 = jnp.sum(ec1, axis=1, keepdims=True) / _F
    cen = ec1 - m2
    var = jnp.sum(cen * cen, axis=1, keepdims=True) / (_F - 1)
    std = jnp.sqrt(var)

    scores = (1.0 - inner) + (ec1 / std) * _R

    iota_f = jax.lax.broadcasted_iota(jnp.int32, (_VB, _F), 1).astype(jnp.float32)
    lane16 = jax.lax.broadcasted_iota(jnp.int32, (_VB, 16), 1)
    out = jnp.zeros((_VB, 16), jnp.int32)
    def _rmin(x):
        p = [jnp.min(x[:, i * (_F // 4):(i + 1) * (_F // 4)], axis=1, keepdims=True)
             for i in range(4)]
        return jnp.minimum(jnp.minimum(p[0], p[1]), jnp.minimum(p[2], p[3]))

    for k in range(_NB):
        rowmin = _rmin(scores)
        cand = jnp.where(scores == rowmin, iota_f, float(_F))
        bidx = _rmin(cand)
        out = jnp.where(lane16 == k, bidx.astype(jnp.int32), out)
        scores = jnp.where(iota_f == bidx, jnp.inf, scores)
    idx_ref[...] = out


def _score_topk(pni, fnT, v, centT):
    grid = (_V // _VB,)
    return pl.pallas_call(
        _score_topk_body,
        grid=grid,
        in_specs=[
            pl.BlockSpec((_VB, 8), lambda i: (i, 0)),
            pl.BlockSpec((8, _F), lambda i: (0, 0)),
            pl.BlockSpec((_VB, 8), lambda i: (i, 0)),
            pl.BlockSpec((8, _F), lambda i: (0, 0)),
        ],
        out_specs=pl.BlockSpec((_VB, 16), lambda i: (i, 0)),
        out_shape=jax.ShapeDtypeStruct((_V, 16), jnp.int32),
    )(pni, fnT, v, centT)


def _stagec_sc(fi_flat, faces_flat, fn_flat, verts_flat, pni_flat):
    mesh = plsc.VectorSubcoreMesh(core_axis_name="c", subcore_axis_name="s",
                                  num_cores=_NC, num_subcores=_NSUB)
    fvec = jax.ShapeDtypeStruct((_V,), jnp.float32)
    ivec = jax.ShapeDtypeStruct((_V,), jnp.int32)

    def body(fi_hbm, fc_hbm, fn_hbm, vt_hbm, pni_hbm,
             dist_hbm, ssq_hbm, cidx_hbm, sign_hbm,
             fi_v, fc_v, fn_v, vt_v, pni_v,
             dist_s, ssq_s, cidx_s, sign_s):
        wid = lax.axis_index("s") * _NC + lax.axis_index("c")
        pltpu.sync_copy(fi_hbm.at[pl.ds(wid * (_VPW * 16), _VPW * 16)], fi_v)
        pltpu.sync_copy(fc_hbm, fc_v)
        pltpu.sync_copy(fn_hbm, fn_v)
        pltpu.sync_copy(vt_hbm, vt_v)
        pltpu.sync_copy(pni_hbm.at[pl.ds(wid * (_VPW * 3), _VPW * 3)], pni_v)

        iota16 = lax.broadcasted_iota(jnp.int32, (16,), 0)
        eps = 1e-12

        def _vert_coords(vi):
            b = vi * 3
            return (plsc.load_gather(vt_v, [b]),
                    plsc.load_gather(vt_v, [b + 1]),
                    plsc.load_gather(vt_v, [b + 2]))

        def _tri_sq(px, py, pz, f):
            f3 = f * 3
            va = plsc.load_gather(fc_v, [f3])
            vb = plsc.load_gather(fc_v, [f3 + 1])
            vc = plsc.load_gather(fc_v, [f3 + 2])
            ax, ay, az = _vert_coords(va)
            bx, by, bz = _vert_coords(vb)
            cx, cy, cz = _vert_coords(vc)
            abx, aby, abz = bx - ax, by - ay, bz - az
            acx, acy, acz = cx - ax, cy - ay, cz - az
            apx, apy, apz = px - ax, py - ay, pz - az
            d1 = (abx * apx + aby * apy) + abz * apz
            d2 = (acx * apx + acy * apy) + acz * apz
            bpx, bpy, bpz = px - bx, py - by, pz - bz
            d3 = (abx * bpx + aby * bpy) + abz * bpz
            d4 = (acx * bpx + acy * bpy) + acz * bpz
            cpx, cpy, cpz = px - cx, py - cy, pz - cz
            d5 = (abx * cpx + aby * cpy) + abz * cpz
            d6 = (acx * cpx + acy * cpy) + acz * cpz
            va_ = d3 * d6 - d5 * d4
            vb_ = d5 * d2 - d1 * d6
            vc_ = d1 * d4 - d3 * d2

            def _safe_div(num, den):
                den = jnp.where(jnp.abs(den) < eps, eps, den)
                return num / den

            def _clip01(x):
                return jnp.minimum(jnp.maximum(x, 0.0), 1.0)

            t_ab = _clip01(_safe_div(d1, d1 - d3))
            t_ac = _clip01(_safe_div(d2, d2 - d6))
            t_bc = _clip01(_safe_div(d4 - d3, (d4 - d3) + (d5 - d6)))
            inv = _safe_div(jnp.ones_like(va_), va_ + vb_ + vc_)
            vv = vb_ * inv
            ww = vc_ * inv
            clx = ax + abx * vv + acx * ww
            cly = ay + aby * vv + acy * ww
            clz = az + abz * vv + acz * ww
            c1 = (va_ <= 0) & ((d4 - d3) >= 0) & ((d5 - d6) >= 0)
            clx = jnp.where(c1, bx + (cx - bx) * t_bc, clx)
            cly = jnp.where(c1, by + (cy - by) * t_bc, cly)
            clz = jnp.where(c1, bz + (cz - bz) * t_bc, clz)
            c2 = (vb_ <= 0) & (d2 >= 0) & (d6 <= 0)
            clx = jnp.where(c2, ax + acx * t_ac, clx)
            cly = jnp.where(c2, ay + acy * t_ac, cly)
            clz = jnp.where(c2, az + acz * t_ac, clz)
            c3 = (vc_ <= 0) & (d1 >= 0) & (d3 <= 0)
            clx = jnp.where(c3, ax + abx * t_ab, clx)
            cly = jnp.where(c3, ay + aby * t_ab, cly)
            clz = jnp.where(c3, az + abz * t_ab, clz)
            c4 = (d6 >= 0) & (d5 <= d6)
            clx = jnp.where(c4, cx, clx)
            cly = jnp.where(c4, cy, cly)
            clz = jnp.where(c4, cz, clz)
            c5 = (d3 >= 0) & (d4 <= d3)
            clx = jnp.where(c5, bx, clx)
            cly = jnp.where(c5, by, cly)
            clz = jnp.where(c5, bz, clz)
            c6 = (d1 <= 0) & (d2 <= 0)
            clx = jnp.where(c6, ax, clx)
            cly = jnp.where(c6, ay, cly)
            clz = jnp.where(c6, az, clz)
            dx, dy, dz = px - clx, py - cly, pz - clz
            return (dx * dx + dy * dy) + dz * dz

        def group(g, carry):
            ip = iota16 * 3 + g * 48
            pn0 = plsc.load_gather(pni_v, [ip])
            pn1 = plsc.load_gather(pni_v, [ip + 1])
            pn2 = plsc.load_gather(pni_v, [ip + 2])
            gid = (wid * _VPW + g * 16) * 3
            iv3 = iota16 * 3 + gid
            px = plsc.load_gather(vt_v, [iv3])
            py = plsc.load_gather(vt_v, [iv3 + 1])
            pz = plsc.load_gather(vt_v, [iv3 + 2])

            ivs, fidx = [], []
            for j in range(_NB):
                idx = iota16 * 16 + (g * 256 + j)
                fj = plsc.load_gather(fi_v, [idx])
                fj3 = fj * 3
                n0 = plsc.load_gather(fn_v, [fj3])
                n1 = plsc.load_gather(fn_v, [fj3 + 1])
                n2 = plsc.load_gather(fn_v, [fj3 + 2])
                iv = (pn0 * n0 + pn1 * n1) + pn2 * n2
                iv = jnp.where(iv > 0.5, iv, -1.0)
                ivs.append(iv)
                fidx.append(fj)

            mindist = jnp.full((16,), jnp.inf, jnp.float32)
            fstar = jnp.zeros((16,), jnp.int32)
            neg_inf = jnp.full((16,), -jnp.inf, jnp.float32)
            for k in range(_NSEL):
                best = ivs[0]
                bidx = fidx[0]
                barg = jnp.zeros((16,), jnp.int32)
                for j in range(1, _NB):
                    cond = ivs[j] > best
                    best = jnp.where(cond, ivs[j], best)
                    bidx = jnp.where(cond, fidx[j], bidx)
                    barg = jnp.where(cond, j, barg)
                for j in range(_NB):
                    ivs[j] = jnp.where(barg == j, neg_inf, ivs[j])
                sq = _tri_sq(px, py, pz, bidx)
                cond2 = sq < mindist
                mindist = jnp.where(cond2, sq, mindist)
                fstar = jnp.where(cond2, bidx, fstar)

            f3 = fstar * 3
            va = plsc.load_gather(fc_v, [f3])
            vb = plsc.load_gather(fc_v, [f3 + 1])
            vc = plsc.load_gather(fc_v, [f3 + 2])
            ax, ay, az = _vert_coords(va)
            bx, by, bz = _vert_coords(vb)
            cx, cy, cz = _vert_coords(vc)
            n0 = plsc.load_gather(fn_v, [f3])
            n1 = plsc.load_gather(fn_v, [f3 + 1])
            n2 = plsc.load_gather(fn_v, [f3 + 2])
            cenx = ((ax + bx) + cx) / 3.0
            ceny = ((ay + by) + cy) / 3.0
            cenz = ((az + bz) + cz) / 3.0
            dvx, dvy, dvz = px - cenx, py - ceny, pz - cenz
            ssq = (dvx * dvx + dvy * dvy) + dvz * dvz
            sgn = -((dvx * n0 + dvy * n1) + dvz * n2)

            dist_s[pl.ds(g * 16, 16)] = mindist
            ssq_s[pl.ds(g * 16, 16)] = ssq
            cidx_s[pl.ds(g * 16, 16)] = fstar
            sign_s[pl.ds(g * 16, 16)] = sgn
            return carry

        lax.fori_loop(0, _NG, group, 0)
        pltpu.sync_copy(dist_s, dist_hbm.at[pl.ds(wid * _VPW, _VPW)])
        pltpu.sync_copy(ssq_s, ssq_hbm.at[pl.ds(wid * _VPW, _VPW)])
        pltpu.sync_copy(cidx_s, cidx_hbm.at[pl.ds(wid * _VPW, _VPW)])
        pltpu.sync_copy(sign_s, sign_hbm.at[pl.ds(wid * _VPW, _VPW)])

    return pl.kernel(
        body,
        out_type=(fvec, fvec, ivec, fvec),
        mesh=mesh,
        compiler_params=_SC_PARAMS,
        scratch_types=[
            pltpu.VMEM((_VPW * 16,), jnp.int32),
            pltpu.VMEM((_F * 3,), jnp.int32),
            pltpu.VMEM((_F * 3,), jnp.float32),
            pltpu.VMEM((_V * 3,), jnp.float32),
            pltpu.VMEM((_VPW * 3,), jnp.float32),
            pltpu.VMEM((_VPW,), jnp.float32),
            pltpu.VMEM((_VPW,), jnp.float32),
            pltpu.VMEM((_VPW,), jnp.int32),
            pltpu.VMEM((_VPW,), jnp.float32),
        ],
    )(fi_flat, faces_flat, fn_flat, verts_flat, pni_flat)


def kernel(verts, faces):
    faces = faces.astype(jnp.int32)
    verts_p = verts[0]
    faces_flat = faces.reshape(-1)
    verts_flat = verts_p.reshape(-1)

    raw_flat, cent3 = _stagea_sc(faces_flat, verts_flat)
    face_n_raw = raw_flat.reshape(_F, 3)
    faces_normals_packed = face_n_raw / jnp.maximum(
        jnp.linalg.norm(face_n_raw, axis=-1, keepdims=True), 1e-6)
    vn = jnp.zeros_like(verts_p)
    scat_idx = jnp.concatenate([faces[:, 0], faces[:, 1], faces[:, 2]])
    scat_upd = jnp.concatenate([face_n_raw, face_n_raw, face_n_raw])
    vn = vn.at[scat_idx].add(scat_upd)
    pt_normals = vn / jnp.maximum(jnp.linalg.norm(vn, axis=-1, keepdims=True), 1e-6)
    pni3 = -pt_normals

    zpadV = jnp.zeros((_V, 5), jnp.float32)
    zpadF = jnp.zeros((5, _F), jnp.float32)
    pni = jnp.concatenate([pni3, zpadV], axis=1)
    vpad = jnp.concatenate([verts_p, zpadV], axis=1)
    fnT = jnp.concatenate([faces_normals_packed.T, zpadF], axis=0)
    centT = jnp.concatenate([cent3.reshape(3, _F), zpadF], axis=0)
    fi16 = _score_topk(pni, fnT, vpad, centT)

    dist, ssq, closed_indx, sign = _stagec_sc(
        fi16.reshape(-1), faces_flat, faces_normals_packed.reshape(-1),
        verts_flat, pni3.reshape(-1))
    return dist, jnp.sqrt(ssq), closed_indx, sign
